# baseline (speedup 1.0000x reference)
import functools

import jax
import jax.numpy as jnp
from jax import lax
from jax.experimental import pallas as pl
from jax.experimental.pallas import tpu as pltpu
from jax.experimental.pallas import tpu_sc as plsc

N_PTS = 10000
NPAD = 10240
NW = 32
PW = NPAD // NW
PC = 16
NCHUNK = PW // PC
CH = [16, 32, 64, 128, 256, 512]
HS = [448 >> s for s in range(6)]
WS = [256 >> s for s in range(6)]
COFF = [0, 16, 48, 112, 240, 496]
CTOT = 1008
NEW_H, NEW_W = 448.0, 256.0
RATIO_H = 1280.0 / 448.0
RATIO_W = 720.0 / 256.0

VTABS = []
for _s in range(6):
    for _c0 in range(0, CH[_s], 128):
        VTABS.append((_s, _c0, min(128, CH[_s])))


def _inv4_rows(e):
    a = e
    s0 = a[0][0] * a[1][1] - a[1][0] * a[0][1]
    s1 = a[0][0] * a[1][2] - a[1][0] * a[0][2]
    s2 = a[0][0] * a[1][3] - a[1][0] * a[0][3]
    s3 = a[0][1] * a[1][2] - a[1][1] * a[0][2]
    s4 = a[0][1] * a[1][3] - a[1][1] * a[0][3]
    s5 = a[0][2] * a[1][3] - a[1][2] * a[0][3]
    c0 = a[2][0] * a[3][1] - a[3][0] * a[2][1]
    c1 = a[2][0] * a[3][2] - a[3][0] * a[2][2]
    c2 = a[2][0] * a[3][3] - a[3][0] * a[2][3]
    c3 = a[2][1] * a[3][2] - a[3][1] * a[2][2]
    c4 = a[2][1] * a[3][3] - a[3][1] * a[2][3]
    c5 = a[2][2] * a[3][3] - a[3][2] * a[2][3]
    det = s0 * c5 - s1 * c4 + s2 * c3 + s3 * c2 - s4 * c1 + s5 * c0
    r = 1.0 / det
    inv = [[None] * 4 for _ in range(3)]
    inv[0][0] = (a[1][1] * c5 - a[1][2] * c4 + a[1][3] * c3) * r
    inv[0][1] = (-a[0][1] * c5 + a[0][2] * c4 - a[0][3] * c3) * r
    inv[0][2] = (a[3][1] * s5 - a[3][2] * s4 + a[3][3] * s3) * r
    inv[0][3] = (-a[2][1] * s5 + a[2][2] * s4 - a[2][3] * s3) * r
    inv[1][0] = (-a[1][0] * c5 + a[1][2] * c2 - a[1][3] * c1) * r
    inv[1][1] = (a[0][0] * c5 - a[0][2] * c2 + a[0][3] * c1) * r
    inv[1][2] = (-a[3][0] * s5 + a[3][2] * s2 - a[3][3] * s1) * r
    inv[1][3] = (a[2][0] * s5 - a[2][2] * s2 + a[2][3] * s1) * r
    inv[2][0] = (a[1][0] * c4 - a[1][1] * c2 + a[1][3] * c0) * r
    inv[2][1] = (-a[0][0] * c4 + a[0][1] * c2 - a[0][3] * c0) * r
    inv[2][2] = (a[3][0] * s4 - a[3][1] * s2 + a[3][3] * s0) * r
    inv[2][3] = (-a[2][0] * s4 + a[2][1] * s2 - a[2][3] * s0) * r
    return inv


def _bf(v):
    return v.astype(jnp.bfloat16).astype(jnp.float32)


TEX_PER_ROW = [128 // c if c < 128 else 1 for c in CH]


def _prep_body(ext_ref, cams_ref, intr_ref, pts_ref, idx_ref, wts_ref,
               shf_ref):
    e0 = [[ext_ref[0, 4 * j + k] for k in range(4)] for j in range(4)]
    ai = _inv4_rows(e0)
    x = _bf(pts_ref[0:8, :])
    y = _bf(pts_ref[8:16, :])
    z = _bf(pts_ref[16:24, :])
    po = [x * _bf(ai[k][0]) + y * _bf(ai[k][1]) + z * _bf(ai[k][2])
          + _bf(ai[k][3]) for k in range(3)]
    pob = [_bf(p) for p in po]
    hupper = NEW_H - 1.0
    wupper = NEW_W - 1.0
    for i in range(3):
        v = cams_ref[i]
        ev = [[_bf(ext_ref[v, 4 * j + k]) for k in range(4)] for j in range(3)]
        xc = pob[0] * ev[0][0] + pob[1] * ev[0][1] + pob[2] * ev[0][2] + ev[0][3]
        yc = pob[0] * ev[1][0] + pob[1] * ev[1][1] + pob[2] * ev[1][2] + ev[1][3]
        zc = pob[0] * ev[2][0] + pob[1] * ev[2][1] + pob[2] * ev[2][2] + ev[2][3]
        zz = zc + 1e-8
        fxv = intr_ref[0, v]
        fyv = intr_ref[1, v]
        pxv = intr_ref[2, v]
        pyv = intr_ref[3, v]
        h = fyv / RATIO_H * jnp.divide(-yc, -zz) + pyv / RATIO_H
        w = fxv / RATIO_W * jnp.divide(xc, -zz) + pxv / RATIO_W
        h = jnp.where(jnp.isnan(h), hupper, h)
        w = jnp.where(jnp.isnan(w), wupper, w)
        h = jnp.clip(h, 0.0, hupper)
        w = jnp.clip(w, 0.0, wupper)
        for s in range(6):
            inv_sc = 1.0 / float(2 ** s)
            xs = h * inv_sc
            ys = w * inv_sc
            x1 = jnp.clip(jnp.floor(xs).astype(jnp.int32), 0, HS[s] - 1)
            x2 = jnp.clip(jnp.ceil(xs).astype(jnp.int32), 0, HS[s] - 1)
            y1 = jnp.clip(jnp.floor(ys).astype(jnp.int32), 0, WS[s] - 1)
            y2 = jnp.clip(jnp.ceil(ys).astype(jnp.int32), 0, WS[s] - 1)
            x1f = x1.astype(jnp.float32)
            x2f = x2.astype(jnp.float32)
            y1f = y1.astype(jnp.float32)
            y2f = y2.astype(jnp.float32)
            w1 = (x2f - xs) * (y2f - ys)
            w2 = (xs - x1f) * (y2f - ys)
            w3 = (x2f - xs) * (ys - y1f)
            w4 = (xs - x1f) * (ys - y1f)
            base = i * HS[s] * WS[s]
            r0 = ((s * 3 + i) * 4) * 8
            tpr = TEX_PER_ROW[s]
            lg = tpr.bit_length() - 1
            tex = [base + x1 * WS[s] + y1, base + x2 * WS[s] + y1,
                   base + x1 * WS[s] + y2, base + x2 * WS[s] + y2]
            for k in range(4):
                idx_ref[r0 + 8 * k:r0 + 8 * (k + 1), :] = \
                    jnp.right_shift(tex[k], lg)
                shf_ref[r0 + 8 * k:r0 + 8 * (k + 1), :] = \
                    (tex[k] & (tpr - 1)) * CH[s] if tpr > 1 else \
                    jnp.zeros_like(tex[k])
            wts_ref[r0 + 0:r0 + 8, :] = w1
            wts_ref[r0 + 8:r0 + 16, :] = w2
            wts_ref[r0 + 16:r0 + 24, :] = w3
            wts_ref[r0 + 24:r0 + 32, :] = w4


def _run_prep(ext_r, cams, intr, pts24):
    out = pl.pallas_call(
        _prep_body,
        out_shape=[
            jax.ShapeDtypeStruct((576, NPAD // 8), jnp.int32),
            jax.ShapeDtypeStruct((576, NPAD // 8), jnp.float32),
            jax.ShapeDtypeStruct((576, NPAD // 8), jnp.int32),
        ],
        in_specs=[
            pl.BlockSpec(memory_space=pltpu.SMEM),
            pl.BlockSpec(memory_space=pltpu.SMEM),
            pl.BlockSpec(memory_space=pltpu.SMEM),
            pl.BlockSpec(memory_space=pltpu.VMEM),
        ],
        out_specs=[
            pl.BlockSpec(memory_space=pltpu.VMEM),
            pl.BlockSpec(memory_space=pltpu.VMEM),
            pl.BlockSpec(memory_space=pltpu.VMEM),
        ],
    )(ext_r, cams, intr, pts24)
    idx72 = out[0].reshape(72, NPAD)
    wts72 = out[1].reshape(72, NPAD)
    shf72 = out[2].reshape(72, NPAD)
    return idx72, wts72, shf72


def _sqrt16(a):
    i = plsc.bitcast(a, jnp.int32)
    i = jnp.int32(0x5F3759DF) - jnp.right_shift(i, 1)
    y = plsc.bitcast(i, jnp.float32)
    y = y * (1.5 - 0.5 * a * y * y)
    y = y * (1.5 - 0.5 * a * y * y)
    return a * y


def _sc_body(t0, t1, t2, t3, t4a, t4b, t5a, t5b, t5c, t5d,
             idx_hbm, wts_hbm, shf_hbm, out_hbm,
             pool, out_buf, idx_v, wts_v, shf_v, sem0, sem1, sem2):
    tabs = [t0, t1, t2, t3, t4a, t4b, t5a, t5b, t5c, t5d]
    sems = [sem0, sem1, sem2]
    wid = lax.axis_index("s") * 2 + lax.axis_index("c")
    lio = lax.iota(jnp.int32, PC)
    lios = [lio + 16 * k for k in range(4)]
    svs = [jnp.full((PC,), j, jnp.int32) for j in range(9)]

    def fire(ti):
        s, _, _ = VTABS[ti]
        grp = ti % 3
        return [
            pltpu.async_copy(tabs[ti].at[idx_v.at[s * 3 + i]],
                             pool.at[grp * 3 + i], sems[grp])
            for i in range(3)
        ]

    @pl.loop(0, NCHUNK)
    def _chunk(ch):
        g = wid * NCHUNK + ch
        pltpu.sync_copy(idx_hbm.at[:, pl.ds(g * 64, 64)], idx_v)
        pltpu.sync_copy(wts_hbm.at[:, pl.ds(g * 64, 64)], wts_v)
        pltpu.sync_copy(shf_hbm.at[:, pl.ds(g * 64, 64)], shf_v)
        cops = {0: fire(0), 1: fire(1), 2: fire(2)}
        for ti, (s, c0, cb) in enumerate(VTABS):
            for cp_ in cops[ti]:
                cp_.wait()
            grp = ti % 3
            t18 = s * 3
            wv = [[wts_v[t18 + i, 16 * k:16 * (k + 1)] for k in range(4)]
                  for i in range(3)]
            narrow = cb < 128
            if narrow:
                sh = [[shf_v[t18 + i, 16 * k:16 * (k + 1)] for k in range(4)]
                      for i in range(3)]
            colbase = COFF[s] + c0

            @plsc.parallel_loop(0, cb, unroll=4)
            def _chan(c):
                cv = jnp.full((PC,), c, jnp.int32)
                bl = []
                for i in range(3):
                    sv = svs[grp * 3 + i]
                    if narrow:
                        cols = [sh[i][k] + cv for k in range(4)]
                    else:
                        cols = [cv] * 4
                    q11 = plsc.load_gather(pool, [sv, lios[0], cols[0]])
                    q21 = plsc.load_gather(pool, [sv, lios[1], cols[1]])
                    q12 = plsc.load_gather(pool, [sv, lios[2], cols[2]])
                    q22 = plsc.load_gather(pool, [sv, lios[3], cols[3]])
                    bl.append(wv[i][0] * q11 + wv[i][1] * q21
                              + wv[i][2] * q12 + wv[i][3] * q22)
                b0, b1, b2 = bl
                mx = jnp.maximum(jnp.maximum(b0, b1), b2)
                sm = b0 + b1 + b2
                sq = b0 * b0 + b1 * b1 + b2 * b2
                mean = sm * (1.0 / 3.0)
                var = jnp.maximum(sq * (1.0 / 3.0) - mean * mean, 0.0)
                std = _sqrt16(var)
                cvcol = jnp.full((PC,), colbase, jnp.int32) + cv
                plsc.store_scatter(out_buf, [lio, cvcol], mx)
                plsc.store_scatter(out_buf, [lio, cvcol + CTOT], mean)
                plsc.store_scatter(out_buf, [lio, cvcol + 2 * CTOT], std)

            if ti + 3 < len(VTABS):
                cops[ti + 3] = fire(ti + 3)

        pltpu.sync_copy(out_buf, out_hbm.at[pl.ds(g * PC, PC)])


def _run_sc(tabs, idx18, wts18, shf18):
    mesh = plsc.VectorSubcoreMesh(core_axis_name="c", subcore_axis_name="s")
    cp = pltpu.CompilerParams(needs_layout_passes=False,
                              use_tc_tiling_on_sc=False)
    kern = pl.kernel(
        _sc_body,
        out_type=jax.ShapeDtypeStruct((NPAD, 3 * CTOT), jnp.float32),
        mesh=mesh,
        scratch_types=[
            pltpu.VMEM((9, 64, 128), jnp.float32),
            pltpu.VMEM((PC, 3 * CTOT), jnp.float32),
            pltpu.VMEM((18, 64), jnp.int32),
            pltpu.VMEM((18, 64), jnp.float32),
            pltpu.VMEM((18, 64), jnp.int32),
            pltpu.SemaphoreType.DMA,
            pltpu.SemaphoreType.DMA,
            pltpu.SemaphoreType.DMA,
        ],
        compiler_params=cp,
    )
    return kern(*tabs, idx18, wts18, shf18)


def _make_tables(feats):
    tabs = []
    for s, f in enumerate(feats):
        t = jnp.transpose(f, (0, 2, 3, 1)).reshape(3 * HS[s] * WS[s], CH[s])
        if CH[s] < 128:
            tabs.append(t.reshape(-1, 128))
        elif CH[s] == 128:
            tabs.append(t)
        else:
            for c0 in range(0, CH[s], 128):
                tabs.append(t[:, c0:c0 + 128])
    return tabs


def kernel(inputs, img_feat0, img_feat1, img_feat2, img_feat3, img_feat4,
           img_feat5, sample_views, ext, fx, fy, px, py):
    pts = jnp.transpose(inputs)
    pts = jnp.pad(pts, ((0, 0), (0, NPAD - N_PTS)))
    pts24 = pts.reshape(24, NPAD // 8)
    ext_r = ext.reshape(8, 16)
    intr = jnp.stack([fx, fy, px, py])
    idx72, wts72, shf72 = _run_prep(ext_r, sample_views, intr, pts24)

    def relayout(a):
        return a.reshape(18, 4, NPAD // PC, PC).transpose(0, 2, 1, 3) \
            .reshape(18, NPAD * 4)

    tabs = _make_tables([img_feat0, img_feat1, img_feat2, img_feat3,
                         img_feat4, img_feat5])
    out = _run_sc(tabs, relayout(idx72), relayout(wts72), relayout(shf72))
    return out[:N_PTS]


# speedup vs baseline: 1.0970x; 1.0970x over previous
import functools

import jax
import jax.numpy as jnp
from jax import lax
from jax.experimental import pallas as pl
from jax.experimental.pallas import tpu as pltpu
from jax.experimental.pallas import tpu_sc as plsc

N_PTS = 10000
NPAD = 10240
NW = 32
PW = NPAD // NW
PC = 16
NCHUNK = PW // PC
CH = [16, 32, 64, 128, 256, 512]
HS = [448 >> s for s in range(6)]
WS = [256 >> s for s in range(6)]
COFF = [0, 16, 48, 112, 240, 496]
CTOT = 1008
NEW_H, NEW_W = 448.0, 256.0
RATIO_H = 1280.0 / 448.0
RATIO_W = 720.0 / 256.0

VTABS = []
for _s in range(6):
    for _c0 in range(0, CH[_s], 128):
        VTABS.append((_s, _c0, min(128, CH[_s])))


def _inv4_rows(e):
    a = e
    s0 = a[0][0] * a[1][1] - a[1][0] * a[0][1]
    s1 = a[0][0] * a[1][2] - a[1][0] * a[0][2]
    s2 = a[0][0] * a[1][3] - a[1][0] * a[0][3]
    s3 = a[0][1] * a[1][2] - a[1][1] * a[0][2]
    s4 = a[0][1] * a[1][3] - a[1][1] * a[0][3]
    s5 = a[0][2] * a[1][3] - a[1][2] * a[0][3]
    c0 = a[2][0] * a[3][1] - a[3][0] * a[2][1]
    c1 = a[2][0] * a[3][2] - a[3][0] * a[2][2]
    c2 = a[2][0] * a[3][3] - a[3][0] * a[2][3]
    c3 = a[2][1] * a[3][2] - a[3][1] * a[2][2]
    c4 = a[2][1] * a[3][3] - a[3][1] * a[2][3]
    c5 = a[2][2] * a[3][3] - a[3][2] * a[2][3]
    det = s0 * c5 - s1 * c4 + s2 * c3 + s3 * c2 - s4 * c1 + s5 * c0
    r = 1.0 / det
    inv = [[None] * 4 for _ in range(3)]
    inv[0][0] = (a[1][1] * c5 - a[1][2] * c4 + a[1][3] * c3) * r
    inv[0][1] = (-a[0][1] * c5 + a[0][2] * c4 - a[0][3] * c3) * r
    inv[0][2] = (a[3][1] * s5 - a[3][2] * s4 + a[3][3] * s3) * r
    inv[0][3] = (-a[2][1] * s5 + a[2][2] * s4 - a[2][3] * s3) * r
    inv[1][0] = (-a[1][0] * c5 + a[1][2] * c2 - a[1][3] * c1) * r
    inv[1][1] = (a[0][0] * c5 - a[0][2] * c2 + a[0][3] * c1) * r
    inv[1][2] = (-a[3][0] * s5 + a[3][2] * s2 - a[3][3] * s1) * r
    inv[1][3] = (a[2][0] * s5 - a[2][2] * s2 + a[2][3] * s1) * r
    inv[2][0] = (a[1][0] * c4 - a[1][1] * c2 + a[1][3] * c0) * r
    inv[2][1] = (-a[0][0] * c4 + a[0][1] * c2 - a[0][3] * c0) * r
    inv[2][2] = (a[3][0] * s4 - a[3][1] * s2 + a[3][3] * s0) * r
    inv[2][3] = (-a[2][0] * s4 + a[2][1] * s2 - a[2][3] * s0) * r
    return inv


def _bf(v):
    return v.astype(jnp.bfloat16).astype(jnp.float32)


def _prep_body(ext_ref, cams_ref, intr_ref, pts_ref, idx_ref, wts_ref):
    e0 = [[ext_ref[0, 4 * j + k] for k in range(4)] for j in range(4)]
    ai = _inv4_rows(e0)
    x = _bf(pts_ref[0:8, :])
    y = _bf(pts_ref[8:16, :])
    z = _bf(pts_ref[16:24, :])
    po = [x * _bf(ai[k][0]) + y * _bf(ai[k][1]) + z * _bf(ai[k][2])
          + _bf(ai[k][3]) for k in range(3)]
    pob = [_bf(p) for p in po]
    hupper = NEW_H - 1.0
    wupper = NEW_W - 1.0
    for i in range(3):
        v = cams_ref[i]
        ev = [[_bf(ext_ref[v, 4 * j + k]) for k in range(4)] for j in range(3)]
        xc = pob[0] * ev[0][0] + pob[1] * ev[0][1] + pob[2] * ev[0][2] + ev[0][3]
        yc = pob[0] * ev[1][0] + pob[1] * ev[1][1] + pob[2] * ev[1][2] + ev[1][3]
        zc = pob[0] * ev[2][0] + pob[1] * ev[2][1] + pob[2] * ev[2][2] + ev[2][3]
        zz = zc + 1e-8
        fxv = intr_ref[0, v]
        fyv = intr_ref[1, v]
        pxv = intr_ref[2, v]
        pyv = intr_ref[3, v]
        h = fyv / RATIO_H * jnp.divide(-yc, -zz) + pyv / RATIO_H
        w = fxv / RATIO_W * jnp.divide(xc, -zz) + pxv / RATIO_W
        h = jnp.where(jnp.isnan(h), hupper, h)
        w = jnp.where(jnp.isnan(w), wupper, w)
        h = jnp.clip(h, 0.0, hupper)
        w = jnp.clip(w, 0.0, wupper)
        for s in range(6):
            inv_sc = 1.0 / float(2 ** s)
            xs = h * inv_sc
            ys = w * inv_sc
            x1 = jnp.clip(jnp.floor(xs).astype(jnp.int32), 0, HS[s] - 1)
            x2 = jnp.clip(jnp.ceil(xs).astype(jnp.int32), 0, HS[s] - 1)
            y1 = jnp.clip(jnp.floor(ys).astype(jnp.int32), 0, WS[s] - 1)
            y2 = jnp.clip(jnp.ceil(ys).astype(jnp.int32), 0, WS[s] - 1)
            x1f = x1.astype(jnp.float32)
            x2f = x2.astype(jnp.float32)
            y1f = y1.astype(jnp.float32)
            y2f = y2.astype(jnp.float32)
            w1 = (x2f - xs) * (y2f - ys)
            w2 = (xs - x1f) * (y2f - ys)
            w3 = (x2f - xs) * (ys - y1f)
            w4 = (xs - x1f) * (ys - y1f)
            base = i * HS[s] * WS[s]
            r0 = ((s * 3 + i) * 4) * 8
            idx_ref[r0 + 0:r0 + 8, :] = base + x1 * WS[s] + y1
            idx_ref[r0 + 8:r0 + 16, :] = base + x2 * WS[s] + y1
            idx_ref[r0 + 16:r0 + 24, :] = base + x1 * WS[s] + y2
            idx_ref[r0 + 24:r0 + 32, :] = base + x2 * WS[s] + y2
            wts_ref[r0 + 0:r0 + 8, :] = w1
            wts_ref[r0 + 8:r0 + 16, :] = w2
            wts_ref[r0 + 16:r0 + 24, :] = w3
            wts_ref[r0 + 24:r0 + 32, :] = w4


def _run_prep(ext_r, cams, intr, pts24):
    out = pl.pallas_call(
        _prep_body,
        out_shape=[
            jax.ShapeDtypeStruct((576, NPAD // 8), jnp.int32),
            jax.ShapeDtypeStruct((576, NPAD // 8), jnp.float32),
        ],
        in_specs=[
            pl.BlockSpec(memory_space=pltpu.SMEM),
            pl.BlockSpec(memory_space=pltpu.SMEM),
            pl.BlockSpec(memory_space=pltpu.SMEM),
            pl.BlockSpec(memory_space=pltpu.VMEM),
        ],
        out_specs=[
            pl.BlockSpec(memory_space=pltpu.VMEM),
            pl.BlockSpec(memory_space=pltpu.VMEM),
        ],
    )(ext_r, cams, intr, pts24)
    idx72 = out[0].reshape(72, NPAD)
    wts72 = out[1].reshape(72, NPAD)
    return idx72, wts72


def _sqrt16(a):
    i = plsc.bitcast(a, jnp.int32)
    i = jnp.int32(0x5F3759DF) - jnp.right_shift(i, 1)
    y = plsc.bitcast(i, jnp.float32)
    y = y * (1.5 - 0.5 * a * y * y)
    y = y * (1.5 - 0.5 * a * y * y)
    return a * y


_SLOTS = [("b16", 0), ("b32", 0), ("b64", 0), ("b128", 0), ("b128", 3),
          ("b128", 0), ("b128", 3), ("b128", 0), ("b128", 3), ("b128", 0)]


def _sc_body(t0, t1, t2, t3, t4a, t4b, t5a, t5b, t5c, t5d,
             idx_hbm, wts_hbm, out_hbm,
             b16, b32, b64, b128, out_buf, idx_v, wts_v,
             sem0, sem1, semt0, semt1, semt2, sems_slab):
    tabs = [t0, t1, t2, t3, t4a, t4b, t5a, t5b, t5c, t5d]
    bufcls = {"b16": b16, "b32": b32, "b64": b64, "b128": b128}
    sems = [sem0, sem1]
    smallsems = [semt0, semt1, semt2]
    wid = lax.axis_index("s") * 2 + lax.axis_index("c")
    lio = lax.iota(jnp.int32, PC)
    lios = [lio + 16 * k for k in range(4)]
    g0 = wid * NCHUNK
    glast = g0 + NCHUNK - 1

    def sem_of(ti):
        return smallsems[ti] if ti < 3 else sems[ti % 2]

    def descs(ti, pb):
        s, _, _ = VTABS[ti]
        cls, slot = _SLOTS[ti]
        bufs = bufcls[cls]
        return [(tabs[ti].at[idx_v.at[pb, s * 3 + i]], bufs.at[slot + i],
                 sem_of(ti)) for i in range(3)]

    def fire_tab(ti, pb):
        return [pltpu.async_copy(a, b, c) for a, b, c in descs(ti, pb)]

    def wait_tab(ti, pb):
        for a, b, c in descs(ti, pb):
            pltpu.make_async_copy(a, b, c).wait()

    def fire_slab(g, pb):
        pltpu.async_copy(idx_hbm.at[:, pl.ds(g * 64, 64)], idx_v.at[pb],
                         sems_slab)
        pltpu.async_copy(wts_hbm.at[:, pl.ds(g * 64, 64)], wts_v.at[pb],
                         sems_slab)

    def wait_slab(pb):
        pltpu.make_async_copy(idx_hbm.at[:, pl.ds(0, 64)], idx_v.at[pb],
                              sems_slab).wait()
        pltpu.make_async_copy(wts_hbm.at[:, pl.ds(0, 64)], wts_v.at[pb],
                              sems_slab).wait()

    def compute(ti, pb):
        s, c0, cb = VTABS[ti]
        cls, slot = _SLOTS[ti]
        bufs = bufcls[cls]
        t18 = s * 3
        wv = [[wts_v[pb, t18 + i, 16 * k:16 * (k + 1)] for k in range(4)]
              for i in range(3)]
        sv = [jnp.full((PC,), slot + i, jnp.int32) for i in range(3)]
        colbase = COFF[s] + c0

        @plsc.parallel_loop(0, cb, unroll=4)
        def _chan(c):
            cv = jnp.full((PC,), c, jnp.int32)
            bl = []
            for i in range(3):
                q11 = plsc.load_gather(bufs, [sv[i], lios[0], cv])
                q21 = plsc.load_gather(bufs, [sv[i], lios[1], cv])
                q12 = plsc.load_gather(bufs, [sv[i], lios[2], cv])
                q22 = plsc.load_gather(bufs, [sv[i], lios[3], cv])
                bl.append(wv[i][0] * q11 + wv[i][1] * q21
                          + wv[i][2] * q12 + wv[i][3] * q22)
            b0, b1, b2 = bl
            mx = jnp.maximum(jnp.maximum(b0, b1), b2)
            sm = b0 + b1 + b2
            sq = b0 * b0 + b1 * b1 + b2 * b2
            mean = sm * (1.0 / 3.0)
            var = jnp.maximum(sq * (1.0 / 3.0) - mean * mean, 0.0)
            std = _sqrt16(var)
            cvcol = jnp.full((PC,), colbase, jnp.int32) + cv
            plsc.store_scatter(out_buf, [lio, cvcol], mx)
            plsc.store_scatter(out_buf, [lio, cvcol + CTOT], mean)
            plsc.store_scatter(out_buf, [lio, cvcol + 2 * CTOT], std)

    pltpu.sync_copy(idx_hbm.at[:, pl.ds(g0 * 64, 64)], idx_v.at[0])
    pltpu.sync_copy(wts_hbm.at[:, pl.ds(g0 * 64, 64)], wts_v.at[0])
    for ti in range(3):
        fire_tab(ti, 0)

    @pl.loop(0, NCHUNK // 2)
    def _pair(ch2):
        g2 = g0 + 2 * ch2
        for half in range(2):
            g = g2 + half
            pb = half
            nb = 1 - half
            gn = jnp.minimum(g + 1, glast)
            fire_slab(gn, nb)
            cops = {3: fire_tab(3, pb), 4: fire_tab(4, pb)}
            for ti in range(3):
                wait_tab(ti, pb)
                compute(ti, pb)
            wait_slab(nb)
            for ti in range(3):
                fire_tab(ti, nb)
            for ti in range(3, 10):
                if ti + 1 <= 9 and ti + 1 not in cops:
                    cops[ti + 1] = fire_tab(ti + 1, pb)
                for cp in cops[ti]:
                    cp.wait()
                compute(ti, pb)
            pltpu.sync_copy(out_buf, out_hbm.at[pl.ds(g * PC, PC)])

    for ti in range(3):
        wait_tab(ti, 0)


def _run_sc(tabs, idx18, wts18):
    mesh = plsc.VectorSubcoreMesh(core_axis_name="c", subcore_axis_name="s")
    cp = pltpu.CompilerParams(needs_layout_passes=False,
                              use_tc_tiling_on_sc=False)
    kern = pl.kernel(
        _sc_body,
        out_type=jax.ShapeDtypeStruct((NPAD, 3 * CTOT), jnp.float32),
        mesh=mesh,
        scratch_types=[
            pltpu.VMEM((3, 64, 16), jnp.float32),
            pltpu.VMEM((3, 64, 32), jnp.float32),
            pltpu.VMEM((3, 64, 64), jnp.float32),
            pltpu.VMEM((6, 64, 128), jnp.float32),
            pltpu.VMEM((PC, 3 * CTOT), jnp.float32),
            pltpu.VMEM((2, 18, 64), jnp.int32),
            pltpu.VMEM((2, 18, 64), jnp.float32),
            pltpu.SemaphoreType.DMA,
            pltpu.SemaphoreType.DMA,
            pltpu.SemaphoreType.DMA,
            pltpu.SemaphoreType.DMA,
            pltpu.SemaphoreType.DMA,
            pltpu.SemaphoreType.DMA,
        ],
        compiler_params=cp,
    )
    return kern(*tabs, idx18, wts18)


def _make_tables(feats):
    tabs = []
    for s, f in enumerate(feats):
        t = jnp.transpose(f, (0, 2, 3, 1)).reshape(3 * HS[s] * WS[s], CH[s])
        if CH[s] <= 128:
            tabs.append(t)
        else:
            for c0 in range(0, CH[s], 128):
                tabs.append(t[:, c0:c0 + 128])
    return tabs


def kernel(inputs, img_feat0, img_feat1, img_feat2, img_feat3, img_feat4,
           img_feat5, sample_views, ext, fx, fy, px, py):
    pts = jnp.transpose(inputs)
    pts = jnp.pad(pts, ((0, 0), (0, NPAD - N_PTS)))
    pts24 = pts.reshape(24, NPAD // 8)
    ext_r = ext.reshape(8, 16)
    intr = jnp.stack([fx, fy, px, py])
    idx72, wts72 = _run_prep(ext_r, sample_views, intr, pts24)

    def relayout(a):
        return a.reshape(18, 4, NPAD // PC, PC).transpose(0, 2, 1, 3) \
            .reshape(18, NPAD * 4)

    tabs = _make_tables([img_feat0, img_feat1, img_feat2, img_feat3,
                         img_feat4, img_feat5])
    out = _run_sc(tabs, relayout(idx72), relayout(wts72))
    return out[:N_PTS]
